# baseline (speedup 1.0000x reference)
import functools

import jax
import jax.numpy as jnp
from jax import lax
from jax.experimental import pallas as pl
from jax.experimental.pallas import tpu as pltpu
from jax.experimental.pallas import tpu_sc as plsc

NUM_USERS = 2000
NUM_ITEMS = 8000
EMBED_DIM = 128
N_LAYERS = 3
N_NODES = NUM_USERS + NUM_ITEMS
N_EDGES = 320000

NC = 2
NS = 16
L = 16
NW = NC * NS

CHUNK = 80
CPT = 125

ROWS_PER_SUB = 624
TAIL_ROWS = N_NODES - ROWS_PER_SUB * NS


def _sc_layer(table, adj, wp):
  mesh = plsc.VectorSubcoreMesh(core_axis_name="c", subcore_axis_name="s")

  @functools.partial(
      pl.kernel,
      out_type=jax.ShapeDtypeStruct((NC, N_NODES, EMBED_DIM), jnp.float32),
      mesh=mesh,
      compiler_params=pltpu.CompilerParams(needs_layout_passes=False, use_tc_tiling_on_sc=False),
      scratch_types=[
          pltpu.VMEM((CPT * CHUNK,), jnp.int32),
          pltpu.VMEM((CPT, CHUNK), jnp.int32),
          pltpu.VMEM((CHUNK,), jnp.float32),
          pltpu.VMEM((CHUNK,), jnp.float32),
          pltpu.VMEM((CHUNK, EMBED_DIM // 2), jnp.int32),
          pltpu.VMEM((CHUNK, EMBED_DIM // 2), jnp.int32),
          pltpu.VMEM((CHUNK, EMBED_DIM), jnp.float32),
          pltpu.VMEM_SHARED((N_NODES, EMBED_DIM), jnp.float32),
          pltpu.SemaphoreType.DMA,
          pltpu.SemaphoreType.DMA,
      ],
  )
  def k(table_h, adj_h, w_h, out_h, src_all, dst_all, w0, w1, rows0,
        rows1, fbuf, acc_sh, sem0, sem1):
    c = lax.axis_index("c")
    s = lax.axis_index("s")
    wid = s * NC + c
    eb = wid * (CPT * CHUNK)

    def dpre(i, _):
      o = pl.ds(eb + i * CHUNK, CHUNK)
      v = pl.ds(i * CHUNK, CHUNK)
      pltpu.async_copy(adj_h.at[pl.ds(N_EDGES + eb + i * CHUNK, CHUNK)], src_all.at[v], sem1)
      pltpu.async_copy(adj_h.at[o], dst_all.at[i], sem1)
      return 0

    lax.fori_loop(0, CPT, dpre, 0)

    zeros16 = jnp.zeros((L,), jnp.float32)

    def zbody(i, _):
      for d in range(EMBED_DIM // L):
        fbuf[i, pl.ds(d * L, L)] = zeros16
      return 0

    lax.fori_loop(0, CHUNK, zbody, 0)
    for z in range(ROWS_PER_SUB // CHUNK):
      pltpu.sync_copy(fbuf,
                      acc_sh.at[pl.ds(s * ROWS_PER_SUB + z * CHUNK, CHUNK)])
    ztail = ROWS_PER_SUB - (ROWS_PER_SUB // CHUNK) * CHUNK
    if ztail:
      pltpu.sync_copy(
          fbuf.at[pl.ds(0, ztail)],
          acc_sh.at[pl.ds(s * ROWS_PER_SUB + ROWS_PER_SUB - ztail, ztail)])

    @pl.when(s == 0)
    def _():
      pltpu.sync_copy(fbuf.at[pl.ds(0, TAIL_ROWS)],
                      acc_sh.at[pl.ds(ROWS_PER_SUB * NS, TAIL_ROWS)])

    def ddrain(i, _):
      o = pl.ds(eb + i * CHUNK, CHUNK)
      v = pl.ds(i * CHUNK, CHUNK)
      pltpu.make_async_copy(adj_h.at[pl.ds(N_EDGES + eb + i * CHUNK, CHUNK)], src_all.at[v], sem1).wait()
      pltpu.make_async_copy(adj_h.at[o], dst_all.at[i], sem1).wait()
      return 0

    lax.fori_loop(0, CPT, ddrain, 0)
    plsc.subcore_barrier()

    rows = (rows0, rows1)
    wbufs = (w0, w1)
    gsems = (sem0, sem1)

    def gather_start(ci, b):
      pltpu.async_copy(w_h.at[pl.ds(eb + ci * CHUNK, CHUNK)], wbufs[b],
                       gsems[b])
      pltpu.async_copy(table_h.at[src_all.at[pl.ds(ci * CHUNK, CHUNK)]],
                       rows[b], gsems[b])

    def gather_wait(ci, b):
      pltpu.make_async_copy(w_h.at[pl.ds(eb + ci * CHUNK, CHUNK)], wbufs[b],
                            gsems[b]).wait()
      pltpu.make_async_copy(table_h.at[src_all.at[pl.ds(ci * CHUNK, CHUNK)]],
                            rows[b], gsems[b]).wait()

    def scale_scatter(ci, b):
      rv = rows[b]
      wv = wbufs[b]

      def sbody(g, _):
        wg = wv[pl.ds(g * L, L)]
        for j in range(L):
          e = g * L + j
          wsp = jnp.full((L,), wg[j], jnp.float32)
          for q in range(EMBED_DIM // (2 * L)):
            xi = rv[e, pl.ds(q * L, L)]
            xa = plsc.bitcast(xi << 16, jnp.float32)
            xb = plsc.bitcast(xi & jnp.int32(-65536), jnp.float32)
            fbuf[e, pl.ds(q * 2 * L, L)] = xa * wsp
            fbuf[e, pl.ds(q * 2 * L + L, L)] = xb * wsp
        return 0

      lax.fori_loop(0, CHUNK // L, sbody, 0)
      pltpu.sync_copy(fbuf, acc_sh.at[dst_all.at[ci]], add=True)

    gather_start(0, 0)

    def pair(p, _):
      i0 = 2 * p
      gather_start(i0 + 1, 1)
      gather_wait(i0, 0)
      scale_scatter(i0, 0)
      nxt = jnp.minimum(i0 + 2, CPT - 1)
      gather_start(nxt, 0)
      gather_wait(i0 + 1, 1)
      scale_scatter(i0 + 1, 1)
      return 0

    lax.fori_loop(0, CPT // 2, pair, 0)
    gather_wait(CPT - 1, 0)
    scale_scatter(CPT - 1, 0)

    plsc.subcore_barrier()
    pltpu.sync_copy(acc_sh.at[pl.ds(s * ROWS_PER_SUB, ROWS_PER_SUB)],
                    out_h.at[c, pl.ds(s * ROWS_PER_SUB, ROWS_PER_SUB)])

    @pl.when(s == 0)
    def _():
      pltpu.sync_copy(acc_sh.at[pl.ds(ROWS_PER_SUB * NS, TAIL_ROWS)],
                      out_h.at[c, pl.ds(ROWS_PER_SUB * NS, TAIL_ROWS)])

  return k(table, adj, wp)


_NB = 10
_BLK = N_NODES // _NB


def _bf16_bits(x):
  xi = jax.lax.bitcast_convert_type(x, jnp.int32)
  return (xi + 0x7FFF + ((xi >> 16) & 1)) >> 16


def _interleave_bf16(t):
  r = t.reshape(_BLK, EMBED_DIM // 32, 32)
  lo = _bf16_bits(r[:, :, :16]) & 0xFFFF
  hi = _bf16_bits(r[:, :, 16:]) << 16
  return (hi | lo).reshape(_BLK, EMBED_DIM // 2)


def _next_table(partials):
  def body(p_ref, t_ref):
    t_ref[...] = _interleave_bf16(p_ref[0] + p_ref[1])

  return pl.pallas_call(
      body,
      grid=(_NB,),
      in_specs=[pl.BlockSpec((2, _BLK, EMBED_DIM), lambda i: (0, i, 0))],
      out_specs=pl.BlockSpec((_BLK, EMBED_DIM // 2), lambda i: (i, 0)),
      out_shape=jax.ShapeDtypeStruct((N_NODES, EMBED_DIM // 2), jnp.int32),
  )(partials)


def _prep_table(e0):
  def body(e_ref, t_ref):
    t_ref[...] = _interleave_bf16(e_ref[...])

  return pl.pallas_call(
      body,
      grid=(_NB,),
      in_specs=[pl.BlockSpec((_BLK, EMBED_DIM), lambda i: (i, 0))],
      out_specs=pl.BlockSpec((_BLK, EMBED_DIM // 2), lambda i: (i, 0)),
      out_shape=jax.ShapeDtypeStruct((N_NODES, EMBED_DIM // 2), jnp.int32),
  )(e0)


def _fold(e0, p1, p2, p3):
  def body(e_ref, a_ref, b_ref, c_ref, o_ref):
    o_ref[...] = (e_ref[...] + (a_ref[0] + a_ref[1]) + (b_ref[0] + b_ref[1]) +
                  (c_ref[0] + c_ref[1])) * 0.25

  pspec = pl.BlockSpec((2, _BLK, EMBED_DIM), lambda i: (0, i, 0))
  espec = pl.BlockSpec((_BLK, EMBED_DIM), lambda i: (i, 0))
  return pl.pallas_call(
      body,
      grid=(_NB,),
      in_specs=[espec, pspec, pspec, pspec],
      out_specs=espec,
      out_shape=jax.ShapeDtypeStruct((N_NODES, EMBED_DIM), jnp.float32),
  )(e0, p1, p2, p3)


def kernel(adj_indices, adj_values, user_emb, item_emb):
  all_emb = jnp.concatenate([user_emb, item_emb], axis=0)
  adj = adj_indices.astype(jnp.int32).reshape(-1)

  p1 = _sc_layer(_prep_table(all_emb), adj, adj_values)
  p2 = _sc_layer(_next_table(p1), adj, adj_values)
  p3 = _sc_layer(_next_table(p2), adj, adj_values)
  out = _fold(all_emb, p1, p2, p3)
  return out[:NUM_USERS], out[NUM_USERS:]


# speedup vs baseline: 2.1388x; 2.1388x over previous
import functools

import jax
import jax.numpy as jnp
from jax import lax
from jax.experimental import pallas as pl
from jax.experimental.pallas import tpu as pltpu
from jax.experimental.pallas import tpu_sc as plsc

NUM_USERS = 2000
NUM_ITEMS = 8000
EMBED_DIM = 128
N_LAYERS = 3
N_NODES = NUM_USERS + NUM_ITEMS
N_EDGES = 320000

NC = 2
NS = 16
L = 16
NW = NC * NS

CHUNK = 80
CPT = 125

ROWS_PER_SUB = 624
TAIL_ROWS = N_NODES - ROWS_PER_SUB * NS


def _sc_layer(table, adj, wp):
  mesh = plsc.VectorSubcoreMesh(core_axis_name="c", subcore_axis_name="s")

  @functools.partial(
      pl.kernel,
      out_type=jax.ShapeDtypeStruct((NC, N_NODES, EMBED_DIM), jnp.float32),
      mesh=mesh,
      scratch_types=[
          pltpu.VMEM((CPT * CHUNK,), jnp.int32),
          pltpu.VMEM((CPT, CHUNK), jnp.int32),
          pltpu.VMEM((CHUNK,), jnp.float32),
          pltpu.VMEM((CHUNK,), jnp.float32),
          pltpu.VMEM((CHUNK, EMBED_DIM), jnp.float32),
          pltpu.VMEM((CHUNK, EMBED_DIM), jnp.float32),
          pltpu.VMEM_SHARED((N_NODES, EMBED_DIM), jnp.float32),
          pltpu.SemaphoreType.DMA,
          pltpu.SemaphoreType.DMA,
      ],
  )
  def k(table_h, adj_h, w_h, out_h, src_all, dst_all, w0, w1, rows0,
        rows1, acc_sh, sem0, sem1):
    c = lax.axis_index("c")
    s = lax.axis_index("s")
    wid = s * NC + c
    eb = wid * (CPT * CHUNK)

    def dpre(i, _):
      o = pl.ds(eb + i * CHUNK, CHUNK)
      v = pl.ds(i * CHUNK, CHUNK)
      pltpu.async_copy(adj_h.at[pl.ds(N_EDGES + eb + i * CHUNK, CHUNK)], src_all.at[v], sem1)
      pltpu.async_copy(adj_h.at[o], dst_all.at[i], sem1)
      return 0

    lax.fori_loop(0, CPT, dpre, 0)

    zeros16 = jnp.zeros((L,), jnp.float32)

    def zbody(i, _):
      for d in range(EMBED_DIM // L):
        rows0[i, pl.ds(d * L, L)] = zeros16
      return 0

    lax.fori_loop(0, CHUNK, zbody, 0)
    for z in range(ROWS_PER_SUB // CHUNK):
      pltpu.sync_copy(rows0,
                      acc_sh.at[pl.ds(s * ROWS_PER_SUB + z * CHUNK, CHUNK)])
    ztail = ROWS_PER_SUB - (ROWS_PER_SUB // CHUNK) * CHUNK
    if ztail:
      pltpu.sync_copy(
          rows0.at[pl.ds(0, ztail)],
          acc_sh.at[pl.ds(s * ROWS_PER_SUB + ROWS_PER_SUB - ztail, ztail)])

    @pl.when(s == 0)
    def _():
      pltpu.sync_copy(rows0.at[pl.ds(0, TAIL_ROWS)],
                      acc_sh.at[pl.ds(ROWS_PER_SUB * NS, TAIL_ROWS)])

    def ddrain(i, _):
      o = pl.ds(eb + i * CHUNK, CHUNK)
      v = pl.ds(i * CHUNK, CHUNK)
      pltpu.make_async_copy(adj_h.at[pl.ds(N_EDGES + eb + i * CHUNK, CHUNK)], src_all.at[v], sem1).wait()
      pltpu.make_async_copy(adj_h.at[o], dst_all.at[i], sem1).wait()
      return 0

    lax.fori_loop(0, CPT, ddrain, 0)
    plsc.subcore_barrier()

    rows = (rows0, rows1)
    wbufs = (w0, w1)
    gsems = (sem0, sem1)

    def gather_start(ci, b):
      pltpu.async_copy(w_h.at[pl.ds(eb + ci * CHUNK, CHUNK)], wbufs[b],
                       gsems[b])
      pltpu.async_copy(table_h.at[src_all.at[pl.ds(ci * CHUNK, CHUNK)]],
                       rows[b], gsems[b])

    def gather_wait(ci, b):
      pltpu.make_async_copy(w_h.at[pl.ds(eb + ci * CHUNK, CHUNK)], wbufs[b],
                            gsems[b]).wait()
      pltpu.make_async_copy(table_h.at[src_all.at[pl.ds(ci * CHUNK, CHUNK)]],
                            rows[b], gsems[b]).wait()

    def scale_scatter(ci, b):
      rv = rows[b]
      wv = wbufs[b]

      def sbody(g, _):
        wg = wv[pl.ds(g * L, L)]
        for j in range(L):
          e = g * L + j
          wsp = jnp.full((L,), wg[j], jnp.float32)
          for d in range(EMBED_DIM // L):
            rv[e, pl.ds(d * L, L)] = rv[e, pl.ds(d * L, L)] * wsp
        return 0

      lax.fori_loop(0, CHUNK // L, sbody, 0)
      pltpu.sync_copy(rv, acc_sh.at[dst_all.at[ci]], add=True)

    gather_start(0, 0)

    def pair(p, _):
      i0 = 2 * p
      gather_start(i0 + 1, 1)
      gather_wait(i0, 0)
      scale_scatter(i0, 0)
      nxt = jnp.minimum(i0 + 2, CPT - 1)
      gather_start(nxt, 0)
      gather_wait(i0 + 1, 1)
      scale_scatter(i0 + 1, 1)
      return 0

    lax.fori_loop(0, CPT // 2, pair, 0)
    gather_wait(CPT - 1, 0)
    scale_scatter(CPT - 1, 0)

    plsc.subcore_barrier()
    pltpu.sync_copy(acc_sh.at[pl.ds(s * ROWS_PER_SUB, ROWS_PER_SUB)],
                    out_h.at[c, pl.ds(s * ROWS_PER_SUB, ROWS_PER_SUB)])

    @pl.when(s == 0)
    def _():
      pltpu.sync_copy(acc_sh.at[pl.ds(ROWS_PER_SUB * NS, TAIL_ROWS)],
                      out_h.at[c, pl.ds(ROWS_PER_SUB * NS, TAIL_ROWS)])

  return k(table, adj, wp)


_NB = 10
_BLK = N_NODES // _NB


def _next_table(partials):
  def body(p_ref, t_ref):
    t_ref[...] = p_ref[0] + p_ref[1]

  return pl.pallas_call(
      body,
      grid=(_NB,),
      in_specs=[pl.BlockSpec((2, _BLK, EMBED_DIM), lambda i: (0, i, 0))],
      out_specs=pl.BlockSpec((_BLK, EMBED_DIM), lambda i: (i, 0)),
      out_shape=jax.ShapeDtypeStruct((N_NODES, EMBED_DIM), jnp.float32),
  )(partials)


def _fold(e0, p1, p2, p3):
  def body(e_ref, a_ref, b_ref, c_ref, o_ref):
    o_ref[...] = (e_ref[...] + (a_ref[0] + a_ref[1]) + (b_ref[0] + b_ref[1]) +
                  (c_ref[0] + c_ref[1])) * 0.25

  pspec = pl.BlockSpec((2, _BLK, EMBED_DIM), lambda i: (0, i, 0))
  espec = pl.BlockSpec((_BLK, EMBED_DIM), lambda i: (i, 0))
  return pl.pallas_call(
      body,
      grid=(_NB,),
      in_specs=[espec, pspec, pspec, pspec],
      out_specs=espec,
      out_shape=jax.ShapeDtypeStruct((N_NODES, EMBED_DIM), jnp.float32),
  )(e0, p1, p2, p3)


def kernel(adj_indices, adj_values, user_emb, item_emb):
  all_emb = jnp.concatenate([user_emb, item_emb], axis=0)
  adj = adj_indices.astype(jnp.int32).reshape(-1)

  p1 = _sc_layer(all_emb, adj, adj_values)
  p2 = _sc_layer(_next_table(p1), adj, adj_values)
  p3 = _sc_layer(_next_table(p2), adj, adj_values)
  out = _fold(all_emb, p1, p2, p3)
  return out[:NUM_USERS], out[NUM_USERS:]
